# baseline (speedup 1.0000x reference)
import functools

import jax
import jax.numpy as jnp
from jax import lax
from jax.experimental import pallas as pl
from jax.experimental.pallas import tpu as pltpu
from jax.experimental.pallas import tpu_sc as plsc

N = 10000
E = 320000
D = 128

NC = 2
NS = 16
NW = NC * NS
CE = 80
CH = 128
EPW = CH * CE
EP = NW * EPW
NBUF = 4
NBLK = CH // NBUF
HALF = NBUF // 2
NP = 10240
SPT = NP // NS
WB = 128

_mesh = plsc.VectorSubcoreMesh(core_axis_name="c", subcore_axis_name="s")


def _deg_body(row_hbm, ew_hbm, deg_out, row_v, ew_v, zbuf, deg_sh,
              dsem0, dsem1, dsem2, dsem3):
    cid = lax.axis_index("c")
    sid = lax.axis_index("s")
    wid = sid * NC + cid
    dsems = [dsem0, dsem1, dsem2, dsem3]

    def _z(i, _):
        zbuf[pl.ds(i * 16, 16)] = jnp.zeros((16,), jnp.float32)
        return 0
    lax.fori_loop(0, SPT // 16, _z, 0)
    pltpu.sync_copy(zbuf, deg_sh.at[pl.ds(sid * SPT, SPT)])
    plsc.subcore_barrier()

    pltpu.sync_copy(row_hbm.at[wid], row_v)
    pltpu.sync_copy(ew_hbm.at[wid], ew_v)

    def _dstart(j, b):
        pltpu.async_copy(ew_v.at[pl.ds(j * CE, CE)],
                         deg_sh.at[row_v.at[j]], dsems[b], add=True)

    def _dwait(j, b):
        pltpu.make_async_copy(ew_v.at[pl.ds(j * CE, CE)],
                              deg_sh.at[row_v.at[j]], dsems[b]).wait()

    for b in range(4):
        _dstart(b, b)

    def _blk(q, _):
        j0 = 4 * q
        for b in range(4):
            _dwait(j0 + b, b)
            _dstart(j0 + 4 + b, b)
        return 0
    lax.fori_loop(0, CH // 4 - 1, _blk, 0)
    for b in range(4):
        _dwait(CH - 4 + b, b)
    plsc.subcore_barrier()

    pltpu.sync_copy(deg_sh.at[pl.ds(sid * SPT, SPT)],
                    deg_out.at[cid, pl.ds(sid * SPT, SPT)])


_deg_kernel = pl.kernel(
    _deg_body,
    out_type=jax.ShapeDtypeStruct((NC, NP), jnp.float32),
    mesh=_mesh,
    scratch_types=[
        pltpu.VMEM((CH, CE), jnp.int32),
        pltpu.VMEM((EPW,), jnp.float32),
        pltpu.VMEM((SPT,), jnp.float32),
        pltpu.VMEM_SHARED((NP,), jnp.float32),
        pltpu.SemaphoreType.DMA,
        pltpu.SemaphoreType.DMA,
        pltpu.SemaphoreType.DMA,
        pltpu.SemaphoreType.DMA,
    ],
    compiler_params=pltpu.CompilerParams(needs_layout_passes=False),
)


def _scale_body(deg_ref, seq_ref, o_ref):
    p = deg_ref[...]
    dinv = lax.rsqrt(p[0] + p[1] + 3.0)
    o_ref[...] = seq_ref[...] * dinv[:, None]


def _scale_kernel(deg_parts, seqp):
    br = 1024
    return pl.pallas_call(
        _scale_body,
        grid=(NP // br,),
        in_specs=[
            pl.BlockSpec((NC, br), lambda i: (0, i)),
            pl.BlockSpec((br, D), lambda i: (i, 0)),
        ],
        out_specs=pl.BlockSpec((br, D), lambda i: (i, 0)),
        out_shape=jax.ShapeDtypeStruct((NP, D), jnp.float32),
    )(deg_parts, seqp)


def _spmm_body(s2_hbm, col_hbm, rep_hbm, out_hbm, *args):
    colbs = args[0:NBUF]
    rebs = args[NBUF:2 * NBUF]
    bufs = args[2 * NBUF:3 * NBUF]
    out_sh = args[3 * NBUF]
    iosems = args[3 * NBUF + 1:3 * NBUF + 1 + NBUF]
    ssems = args[3 * NBUF + 1 + NBUF:3 * NBUF + 1 + 2 * NBUF]

    cid = lax.axis_index("c")
    sid = lax.axis_index("s")
    wid = sid * NC + cid

    def _z(i, _):
        bufs[0][i // 8, pl.ds((i % 8) * 16, 16)] = jnp.zeros((16,),
                                                            jnp.float32)
        return 0
    lax.fori_loop(0, WB * 8, _z, 0)
    for k in range(SPT // WB):
        pltpu.sync_copy(bufs[0].at[pl.ds(0, WB)],
                        out_sh.at[pl.ds(sid * SPT + k * WB, WB)])
    plsc.subcore_barrier()

    def _cstart(j, b):
        pltpu.async_copy(col_hbm.at[wid, j], colbs[b], iosems[b])

    def _cwait(j, b):
        pltpu.make_async_copy(col_hbm.at[wid, j], colbs[b], iosems[b]).wait()

    def _estart(j, b):
        pltpu.async_copy(rep_hbm.at[wid, j], rebs[b], ssems[b])

    def _ewait(j, b):
        pltpu.make_async_copy(rep_hbm.at[wid, j], rebs[b], ssems[b]).wait()

    def _gstart(b):
        pltpu.async_copy(s2_hbm.at[colbs[b]], bufs[b], iosems[b])

    def _gwait(b):
        pltpu.make_async_copy(s2_hbm.at[colbs[b]], bufs[b], iosems[b]).wait()

    def _sstart(b):
        pltpu.async_copy(bufs[b], out_sh.at[rebs[b].at[0]], ssems[b],
                         add=True)

    def _swait(b):
        pltpu.make_async_copy(bufs[b], out_sh.at[rebs[b].at[0]],
                              ssems[b]).wait()

    def _scale(b):
        buf, reb = bufs[b], rebs[b]

        def _grp(g, _):
            ewf = plsc.bitcast(reb[1, pl.ds(g * 16, 16)], jnp.float32)
            for t in range(16):
                w = lax.gather(
                    ewf, jnp.full((16, 1), t, jnp.int32),
                    lax.GatherDimensionNumbers(
                        offset_dims=(), collapsed_slice_dims=(0,),
                        start_index_map=(0,)),
                    (1,), mode=lax.GatherScatterMode.PROMISE_IN_BOUNDS)
                e = g * 16 + t
                for k in range(D // 16):
                    buf[e, pl.ds(k * 16, 16)] = buf[e, pl.ds(k * 16, 16)] * w
            return 0
        lax.fori_loop(0, CE // 16, _grp, 0)

    for b in range(NBUF):
        _cstart(b, b)
        _estart(b, b)
    for b in range(NBUF):
        _cwait(b, b)
        _gstart(b)

    def _blk(q, _):
        j0 = NBUF * q
        for b in range(NBUF):
            jb = j0 + b
            _gwait(b)
            _cstart(jb + NBUF, b)
            _ewait(jb, b)
            _scale(b)
            _sstart(b)
            if b >= HALF:
                bp = b - HALF
                jp = j0 + bp
                _swait(bp)
                _estart(jp + NBUF, bp)
                _cwait(jp + NBUF, bp)
                _gstart(bp)
        for bp in range(NBUF - HALF, NBUF):
            jp = j0 + bp
            _swait(bp)
            _estart(jp + NBUF, bp)
            _cwait(jp + NBUF, bp)
            _gstart(bp)
        return 0
    lax.fori_loop(0, NBLK - 1, _blk, 0)

    j0 = CH - NBUF
    for b in range(NBUF):
        jb = j0 + b
        _gwait(b)
        _ewait(jb, b)
        _scale(b)
        _sstart(b)
    for b in range(NBUF):
        _swait(b)
    plsc.subcore_barrier()

    for k in range(SPT // WB):
        pltpu.sync_copy(out_sh.at[pl.ds(sid * SPT + k * WB, WB)],
                        out_hbm.at[cid, pl.ds(sid * SPT + k * WB, WB)])


_spmm_kernel = pl.kernel(
    _spmm_body,
    out_type=jax.ShapeDtypeStruct((NC, NP, D), jnp.float32),
    mesh=_mesh,
    scratch_types=(
        [pltpu.VMEM((CE,), jnp.int32)] * NBUF
        + [pltpu.VMEM((2, CE), jnp.int32)] * NBUF
        + [pltpu.VMEM((CE, D), jnp.float32)] * NBUF
        + [pltpu.VMEM_SHARED((NP, D), jnp.float32)]
        + [pltpu.SemaphoreType.DMA] * (2 * NBUF)
    ),
    compiler_params=pltpu.CompilerParams(needs_layout_passes=False),
)


def _final_body(deg_ref, parts_ref, s2_ref, w_ref, o_ref):
    p = deg_ref[...]
    dinv = lax.rsqrt(p[0] + p[1] + 3.0)
    acc = parts_ref[0] + parts_ref[1] + 3.0 * s2_ref[...]
    x = acc * dinv[:, None]
    y = lax.dot_general(x, w_ref[...], (((1,), (1,)), ((), ())),
                        preferred_element_type=jnp.float32)
    o_ref[...] = jnp.maximum(y, 0.0)


def _final_kernel(deg_parts, parts, s2, W):
    br = 1024
    return pl.pallas_call(
        _final_body,
        grid=(NP // br,),
        in_specs=[
            pl.BlockSpec((NC, br), lambda i: (0, i)),
            pl.BlockSpec((NC, br, D), lambda i: (0, i, 0)),
            pl.BlockSpec((br, D), lambda i: (i, 0)),
            pl.BlockSpec((D, D), lambda i: (0, 0)),
        ],
        out_specs=pl.BlockSpec((br, D), lambda i: (i, 0)),
        out_shape=jax.ShapeDtypeStruct((NP, D), jnp.float32),
    )(deg_parts, parts, s2, W)


@jax.jit
def kernel(seq, edge_index, edge_weight, W):
    row = edge_index[0]
    col = edge_index[1]
    pad = EP - E
    pad_idx = (jnp.arange(pad, dtype=jnp.int32) % N)
    rowf = jnp.concatenate([row, pad_idx])
    colf = jnp.concatenate([col, pad_idx])
    ewf = jnp.concatenate([edge_weight, jnp.zeros((pad,), jnp.float32)])
    seqp = jnp.pad(seq, ((0, NP - N), (0, 0)))

    rowp = rowf.reshape(NW, CH, CE)
    ewp = ewf.reshape(NW, EPW)
    colp = colf.reshape(NW, CH, CE)
    rep = jnp.stack(
        [rowf.reshape(NW, CH, CE),
         lax.bitcast_convert_type(ewf, jnp.int32).reshape(NW, CH, CE)],
        axis=2)

    deg_parts = _deg_kernel(rowp, ewp)
    s2 = _scale_kernel(deg_parts, seqp)
    parts = _spmm_kernel(s2, colp, rep)
    outp = _final_kernel(deg_parts, parts, s2, W)
    return outp[:N]


# speedup vs baseline: 1.0163x; 1.0163x over previous
import functools

import jax
import jax.numpy as jnp
from jax import lax
from jax.experimental import pallas as pl
from jax.experimental.pallas import tpu as pltpu
from jax.experimental.pallas import tpu_sc as plsc

N = 10000
E = 320000
D = 128

NC = 2
NS = 16
NW = NC * NS
CE = 112
CH = 90
EPW = CH * CE
EP = NW * EPW
NBUF = 3
NBLK = CH // NBUF
HALF = NBUF // 2
NP = 10240
SPT = NP // NS
WB = 128

_mesh = plsc.VectorSubcoreMesh(core_axis_name="c", subcore_axis_name="s")


def _deg_body(row_hbm, ew_hbm, deg_out, row_v, ew_v, zbuf, deg_sh,
              dsem0, dsem1, dsem2, dsem3):
    cid = lax.axis_index("c")
    sid = lax.axis_index("s")
    wid = sid * NC + cid
    dsems = [dsem0, dsem1, dsem2, dsem3]

    def _z(i, _):
        zbuf[pl.ds(i * 16, 16)] = jnp.zeros((16,), jnp.float32)
        return 0
    lax.fori_loop(0, SPT // 16, _z, 0)
    pltpu.sync_copy(zbuf, deg_sh.at[pl.ds(sid * SPT, SPT)])
    plsc.subcore_barrier()

    pltpu.sync_copy(row_hbm.at[wid], row_v)
    pltpu.sync_copy(ew_hbm.at[wid], ew_v)

    def _dstart(j, b):
        pltpu.async_copy(ew_v.at[pl.ds(j * CE, CE)],
                         deg_sh.at[row_v.at[j]], dsems[b], add=True)

    def _dwait(j, b):
        pltpu.make_async_copy(ew_v.at[pl.ds(j * CE, CE)],
                              deg_sh.at[row_v.at[j]], dsems[b]).wait()

    for b in range(2):
        _dstart(b, b)

    def _blk(q, _):
        j0 = 2 * q
        for b in range(2):
            _dwait(j0 + b, b)
            _dstart(j0 + 2 + b, b)
        return 0
    lax.fori_loop(0, CH // 2 - 1, _blk, 0)
    for b in range(2):
        _dwait(CH - 2 + b, b)
    plsc.subcore_barrier()

    pltpu.sync_copy(deg_sh.at[pl.ds(sid * SPT, SPT)],
                    deg_out.at[cid, pl.ds(sid * SPT, SPT)])


_deg_kernel = pl.kernel(
    _deg_body,
    out_type=jax.ShapeDtypeStruct((NC, NP), jnp.float32),
    mesh=_mesh,
    scratch_types=[
        pltpu.VMEM((CH, CE), jnp.int32),
        pltpu.VMEM((EPW,), jnp.float32),
        pltpu.VMEM((SPT,), jnp.float32),
        pltpu.VMEM_SHARED((NP,), jnp.float32),
        pltpu.SemaphoreType.DMA,
        pltpu.SemaphoreType.DMA,
        pltpu.SemaphoreType.DMA,
        pltpu.SemaphoreType.DMA,
    ],
    compiler_params=pltpu.CompilerParams(needs_layout_passes=False),
)


def _scale_body(deg_ref, seq_ref, o_ref):
    p = deg_ref[...]
    dinv = lax.rsqrt(p[0] + p[1] + 3.0)
    o_ref[...] = seq_ref[...] * dinv[:, None]


def _scale_kernel(deg_parts, seqp):
    br = 1024
    return pl.pallas_call(
        _scale_body,
        grid=(NP // br,),
        in_specs=[
            pl.BlockSpec((NC, br), lambda i: (0, i)),
            pl.BlockSpec((br, D), lambda i: (i, 0)),
        ],
        out_specs=pl.BlockSpec((br, D), lambda i: (i, 0)),
        out_shape=jax.ShapeDtypeStruct((NP, D), jnp.float32),
    )(deg_parts, seqp)


def _spmm_body(s2_hbm, col_hbm, rep_hbm, out_hbm, *args):
    colbs = args[0:NBUF]
    rebs = args[NBUF:2 * NBUF]
    bufs = args[2 * NBUF:3 * NBUF]
    out_sh = args[3 * NBUF]
    iosems = args[3 * NBUF + 1:3 * NBUF + 1 + NBUF]
    ssems = args[3 * NBUF + 1 + NBUF:3 * NBUF + 1 + 2 * NBUF]

    cid = lax.axis_index("c")
    sid = lax.axis_index("s")
    wid = sid * NC + cid

    def _z(i, _):
        bufs[0][i // 8, pl.ds((i % 8) * 16, 16)] = jnp.zeros((16,),
                                                            jnp.float32)
        return 0
    lax.fori_loop(0, WB * 8, _z, 0)
    for k in range(SPT // WB):
        pltpu.sync_copy(bufs[0].at[pl.ds(0, WB)],
                        out_sh.at[pl.ds(sid * SPT + k * WB, WB)])
    plsc.subcore_barrier()

    def _cstart(j, b):
        pltpu.async_copy(col_hbm.at[wid, j], colbs[b], iosems[b])

    def _cwait(j, b):
        pltpu.make_async_copy(col_hbm.at[wid, j], colbs[b], iosems[b]).wait()

    def _estart(j, b):
        pltpu.async_copy(rep_hbm.at[wid, j], rebs[b], ssems[b])

    def _ewait(j, b):
        pltpu.make_async_copy(rep_hbm.at[wid, j], rebs[b], ssems[b]).wait()

    def _gstart(b):
        pltpu.async_copy(s2_hbm.at[colbs[b]], bufs[b], iosems[b])

    def _gwait(b):
        pltpu.make_async_copy(s2_hbm.at[colbs[b]], bufs[b], iosems[b]).wait()

    def _sstart(b):
        pltpu.async_copy(bufs[b], out_sh.at[rebs[b].at[0]], ssems[b],
                         add=True)

    def _swait(b):
        pltpu.make_async_copy(bufs[b], out_sh.at[rebs[b].at[0]],
                              ssems[b]).wait()

    def _scale(b):
        buf, reb = bufs[b], rebs[b]

        def _grp(g, _):
            ewf = plsc.bitcast(reb[1, pl.ds(g * 16, 16)], jnp.float32)
            for t in range(16):
                w = lax.gather(
                    ewf, jnp.full((16, 1), t, jnp.int32),
                    lax.GatherDimensionNumbers(
                        offset_dims=(), collapsed_slice_dims=(0,),
                        start_index_map=(0,)),
                    (1,), mode=lax.GatherScatterMode.PROMISE_IN_BOUNDS)
                e = g * 16 + t
                for k in range(D // 16):
                    buf[e, pl.ds(k * 16, 16)] = buf[e, pl.ds(k * 16, 16)] * w
            return 0
        lax.fori_loop(0, CE // 16, _grp, 0)

    for b in range(NBUF):
        _cstart(b, b)
        _estart(b, b)
    for b in range(NBUF):
        _cwait(b, b)
        _gstart(b)

    def _blk(q, _):
        j0 = NBUF * q
        for b in range(NBUF):
            jb = j0 + b
            _gwait(b)
            _cstart(jb + NBUF, b)
            _ewait(jb, b)
            _scale(b)
            _sstart(b)
            if b >= HALF:
                bp = b - HALF
                jp = j0 + bp
                _swait(bp)
                _estart(jp + NBUF, bp)
                _cwait(jp + NBUF, bp)
                _gstart(bp)
        for bp in range(NBUF - HALF, NBUF):
            jp = j0 + bp
            _swait(bp)
            _estart(jp + NBUF, bp)
            _cwait(jp + NBUF, bp)
            _gstart(bp)
        return 0
    lax.fori_loop(0, NBLK - 1, _blk, 0)

    j0 = CH - NBUF
    for b in range(NBUF):
        jb = j0 + b
        _gwait(b)
        _ewait(jb, b)
        _scale(b)
        _sstart(b)
    for b in range(NBUF):
        _swait(b)
    plsc.subcore_barrier()

    for k in range(SPT // WB):
        pltpu.sync_copy(out_sh.at[pl.ds(sid * SPT + k * WB, WB)],
                        out_hbm.at[cid, pl.ds(sid * SPT + k * WB, WB)])


_spmm_kernel = pl.kernel(
    _spmm_body,
    out_type=jax.ShapeDtypeStruct((NC, NP, D), jnp.float32),
    mesh=_mesh,
    scratch_types=(
        [pltpu.VMEM((CE,), jnp.int32)] * NBUF
        + [pltpu.VMEM((2, CE), jnp.int32)] * NBUF
        + [pltpu.VMEM((CE, D), jnp.float32)] * NBUF
        + [pltpu.VMEM_SHARED((NP, D), jnp.float32)]
        + [pltpu.SemaphoreType.DMA] * (2 * NBUF)
    ),
    compiler_params=pltpu.CompilerParams(needs_layout_passes=False),
)


def _final_body(deg_ref, parts_ref, s2_ref, w_ref, o_ref):
    p = deg_ref[...]
    dinv = lax.rsqrt(p[0] + p[1] + 3.0)
    acc = parts_ref[0] + parts_ref[1] + 3.0 * s2_ref[...]
    x = acc * dinv[:, None]
    y = lax.dot_general(x, w_ref[...], (((1,), (1,)), ((), ())),
                        preferred_element_type=jnp.float32)
    o_ref[...] = jnp.maximum(y, 0.0)


def _final_kernel(deg_parts, parts, s2, W):
    br = 1024
    return pl.pallas_call(
        _final_body,
        grid=(NP // br,),
        in_specs=[
            pl.BlockSpec((NC, br), lambda i: (0, i)),
            pl.BlockSpec((NC, br, D), lambda i: (0, i, 0)),
            pl.BlockSpec((br, D), lambda i: (i, 0)),
            pl.BlockSpec((D, D), lambda i: (0, 0)),
        ],
        out_specs=pl.BlockSpec((br, D), lambda i: (i, 0)),
        out_shape=jax.ShapeDtypeStruct((NP, D), jnp.float32),
    )(deg_parts, parts, s2, W)


@jax.jit
def kernel(seq, edge_index, edge_weight, W):
    row = edge_index[0]
    col = edge_index[1]
    pad = EP - E
    pad_idx = (jnp.arange(pad, dtype=jnp.int32) % N)
    rowf = jnp.concatenate([row, pad_idx])
    colf = jnp.concatenate([col, pad_idx])
    ewf = jnp.concatenate([edge_weight, jnp.zeros((pad,), jnp.float32)])
    seqp = jnp.pad(seq, ((0, NP - N), (0, 0)))

    rowp = rowf.reshape(NW, CH, CE)
    ewp = ewf.reshape(NW, EPW)
    colp = colf.reshape(NW, CH, CE)
    rep = jnp.stack(
        [rowf.reshape(NW, CH, CE),
         lax.bitcast_convert_type(ewf, jnp.int32).reshape(NW, CH, CE)],
        axis=2)

    deg_parts = _deg_kernel(rowp, ewp)
    s2 = _scale_kernel(deg_parts, seqp)
    parts = _spmm_kernel(s2, colp, rep)
    outp = _final_kernel(deg_parts, parts, s2, W)
    return outp[:N]
